# Initial kernel scaffold; baseline (speedup 1.0000x reference)
import jax
import jax.numpy as jnp
from jax.experimental import pallas as pl


def kernel(x, edge_index, batch, lin_W, lin_b, gcl1_W, gcl1_b, pool1_p, hgcl0_W, hgcl0_b, pool2_p, att_W, att_b):
    raise NotImplementedError("write your pallas kernel here")


# speedup vs baseline: 13.3656x; 13.3656x over previous
import functools

import jax
import jax.numpy as jnp
from jax import lax
from jax.experimental import pallas as pl
from jax.experimental.pallas import tpu as pltpu
from jax.experimental.pallas import tpu_sc as plsc

N = 10000
NP = 10240
E = 320000
G = 16
RATIO = 0.5
F = 129
FP = 136
NC, NS, L = 2, 16, 16
NT = NC * NS
CH = 128
CPT = 79
EPT = CH * CPT
EP = NT * EPT
RPT = NP // NS


def _mm2_body(x_ref, w1_ref, b1_ref, w2_ref, o_ref):
    h = jnp.maximum(
        jnp.dot(x_ref[...], w1_ref[...], preferred_element_type=jnp.float32)
        + b1_ref[...], 0.0)
    o_ref[...] = jnp.dot(h, w2_ref[...], preferred_element_type=jnp.float32)


def _mm2(x, w1, b1, w2, bm=1024):
    nb = NP // bm
    return pl.pallas_call(
        _mm2_body,
        grid=(nb,),
        in_specs=[
            pl.BlockSpec((bm, FP), lambda i: (i, 0)),
            pl.BlockSpec((FP, 256), lambda i: (0, 0)),
            pl.BlockSpec((1, 256), lambda i: (0, 0)),
            pl.BlockSpec((256, 128), lambda i: (0, 0)),
        ],
        out_specs=pl.BlockSpec((bm, 128), lambda i: (i, 0)),
        out_shape=jax.ShapeDtypeStruct((NP, 128), jnp.float32),
    )(x, w1, b1, w2)


def _scale_a_body(hp_ref, nm_ref, a_ref, sn_ref):
    degraw = jnp.sum(hp_ref[...], axis=0)
    nm = nm_ref[...]
    deg = nm * degraw + nm
    pos = deg > 0
    dis = jnp.where(pos, lax.rsqrt(jnp.maximum(deg, 1e-12)), 0.0)
    a_ref[...] = dis * nm
    sn_ref[...] = jnp.where(pos, 1.0 / jnp.maximum(deg, 1e-12), 0.0) * nm


def _scale_a(histp3, nm2d, bm=16):
    nb = (NP // 128) // bm
    return pl.pallas_call(
        _scale_a_body,
        grid=(nb,),
        in_specs=[
            pl.BlockSpec((NT, bm, 128), lambda i: (0, i, 0)),
            pl.BlockSpec((bm, 128), lambda i: (i, 0)),
        ],
        out_specs=[
            pl.BlockSpec((bm, 128), lambda i: (i, 0)),
            pl.BlockSpec((bm, 128), lambda i: (i, 0)),
        ],
        out_shape=[
            jax.ShapeDtypeStruct((NP // 128, 128), jnp.float32),
            jax.ShapeDtypeStruct((NP // 128, 128), jnp.float32),
        ],
    )(histp3, nm2d)


def _scale_y_body(a_ref, xw_ref, y_ref):
    y_ref[...] = a_ref[...] * xw_ref[...]


def _scale_y(a, xw, bm=1024):
    nb = NP // bm
    return pl.pallas_call(
        _scale_y_body,
        grid=(nb,),
        in_specs=[
            pl.BlockSpec((bm, 1), lambda i: (i, 0)),
            pl.BlockSpec((bm, 128), lambda i: (i, 0)),
        ],
        out_specs=pl.BlockSpec((bm, 128), lambda i: (i, 0)),
        out_shape=jax.ShapeDtypeStruct((NP, 128), jnp.float32),
    )(a, xw)


def _conv_body(nb, ep_ref, a_ref, sn_ref, xw_ref, b_ref, nm_ref, p_ref,
               batch_ref, h_ref, s_ref, t_ref, k_ref, cnt_acc):
    i = pl.program_id(0)

    @pl.when(i == 0)
    def _():
        cnt_acc[...] = jnp.zeros_like(cnt_acc)

    nm = nm_ref[...]
    agg = (ep_ref[0] + ep_ref[1]) * a_ref[...]
    h = jnp.maximum((agg + sn_ref[...] * xw_ref[...] + b_ref[...]) * nm, 0.0)
    h = h * nm
    h_ref[...] = h
    s = jnp.sum(h * p_ref[...], axis=1, keepdims=True)
    sm = jnp.where(nm > 0, s, -1e30)
    s_ref[...] = sm
    t_ref[...] = jnp.tanh(sm)
    oh = (batch_ref[...] == lax.broadcasted_iota(jnp.int32, (1, G), 1)
          ).astype(jnp.float32) * nm
    cnt_acc[...] += jnp.sum(oh, axis=0, keepdims=True)

    @pl.when(i == nb - 1)
    def _():
        k_ref[...] = jnp.ceil(RATIO * cnt_acc[...])


def _conv(ep, a, sn, xw, b, nm, p_unit, batch, bm=1024):
    nb = NP // bm
    return pl.pallas_call(
        functools.partial(_conv_body, nb),
        grid=(nb,),
        in_specs=[
            pl.BlockSpec((2, bm, 128), lambda i: (0, i, 0)),
            pl.BlockSpec((bm, 1), lambda i: (i, 0)),
            pl.BlockSpec((bm, 1), lambda i: (i, 0)),
            pl.BlockSpec((bm, 128), lambda i: (i, 0)),
            pl.BlockSpec((1, 128), lambda i: (0, 0)),
            pl.BlockSpec((bm, 1), lambda i: (i, 0)),
            pl.BlockSpec((1, 128), lambda i: (0, 0)),
            pl.BlockSpec((bm, 1), lambda i: (i, 0)),
        ],
        out_specs=[
            pl.BlockSpec((bm, 128), lambda i: (i, 0)),
            pl.BlockSpec((bm, 1), lambda i: (i, 0)),
            pl.BlockSpec((bm, 1), lambda i: (i, 0)),
            pl.BlockSpec((1, G), lambda i: (0, 0)),
        ],
        out_shape=[
            jax.ShapeDtypeStruct((NP, 128), jnp.float32),
            jax.ShapeDtypeStruct((NP, 1), jnp.float32),
            jax.ShapeDtypeStruct((NP, 1), jnp.float32),
            jax.ShapeDtypeStruct((1, G), jnp.float32),
        ],
        scratch_shapes=[pltpu.VMEM((1, G), jnp.float32)],
    )(ep, a, sn, xw, b, nm, p_unit, batch)


def _rank_body(bm, bn, s_ref, b_ref, al_ref, sc_ref, bc_ref, alc_ref, k_ref,
               kept_ref):
    i = pl.program_id(0)
    sr = s_ref[...]
    br = b_ref[...]
    ar = al_ref[...]
    irow = i * bm + lax.broadcasted_iota(jnp.int32, (bm, 1), 0)
    cnt = jnp.zeros((bm, 1), jnp.float32)
    for cb in range(NP // bn):
        sc = sc_ref[:, cb * bn:(cb + 1) * bn]
        bc = bc_ref[:, cb * bn:(cb + 1) * bn]
        ac = alc_ref[:, cb * bn:(cb + 1) * bn]
        jcol = cb * bn + lax.broadcasted_iota(jnp.int32, (1, bn), 1)
        better = (sc > sr) | ((sc == sr) & (jcol < irow))
        m = better & (bc == br) & (ac > 0)
        cnt = cnt + jnp.sum(m.astype(jnp.float32), axis=1, keepdims=True)
    oh = (br == lax.broadcasted_iota(jnp.int32, (bm, G), 1)).astype(jnp.float32)
    kr = jnp.sum(oh * k_ref[...], axis=1, keepdims=True)
    kept_ref[...] = jnp.where((ar > 0) & (cnt < kr), 1.0, 0.0)


def _rank(s, batch, alive, s_c, b_c, al_c, k, bm=256, bn=512):
    nb = NP // bm
    return pl.pallas_call(
        functools.partial(_rank_body, bm, bn),
        grid=(nb,),
        in_specs=[
            pl.BlockSpec((bm, 1), lambda i: (i, 0)),
            pl.BlockSpec((bm, 1), lambda i: (i, 0)),
            pl.BlockSpec((bm, 1), lambda i: (i, 0)),
            pl.BlockSpec((1, NP), lambda i: (0, 0)),
            pl.BlockSpec((1, NP), lambda i: (0, 0)),
            pl.BlockSpec((1, NP), lambda i: (0, 0)),
            pl.BlockSpec((1, G), lambda i: (0, 0)),
        ],
        out_specs=pl.BlockSpec((bm, 1), lambda i: (i, 0)),
        out_shape=jax.ShapeDtypeStruct((NP, 1), jnp.float32),
    )(s, batch, alive, s_c, b_c, al_c, k)


def _att_body(nb, with_xw, h_ref, t_ref, kept_ref, batch_ref, aw_ref, ab_ref,
              wn_ref, prev_ref, out_ref, xw_ref, m_acc, d_acc, num_acc):
    i = pl.program_id(0)

    @pl.when(i == 0)
    def _():
        m_acc[...] = jnp.full_like(m_acc, -1e30)
        d_acc[...] = jnp.zeros_like(d_acc)
        num_acc[...] = jnp.zeros_like(num_acc)

    kept = kept_ref[...]
    hp = h_ref[...] * t_ref[...] * kept
    if with_xw:
        xw_ref[...] = jnp.dot(hp, wn_ref[...],
                              preferred_element_type=jnp.float32)
    g = jnp.sum(hp * aw_ref[...], axis=1, keepdims=True) + ab_ref[...]
    gm = jnp.where(kept > 0, g, -1e30)
    oh = (batch_ref[...] == lax.broadcasted_iota(jnp.int32, (1, G), 1))
    ohf = oh.astype(jnp.float32)
    bmax = jnp.max(jnp.where(oh, gm, -1e30), axis=0, keepdims=True)
    m_old = m_acc[...]
    m_new = jnp.maximum(m_old, bmax)
    alpha = jnp.exp(m_old - m_new)
    m_acc[...] = m_new
    mn = jnp.sum(ohf * m_new, axis=1, keepdims=True)
    e = jnp.exp(gm - mn) * kept
    d_acc[...] = d_acc[...] * alpha + jnp.sum(ohf * e, axis=0, keepdims=True)
    eye = (lax.broadcasted_iota(jnp.int32, (G, G), 0)
           == lax.broadcasted_iota(jnp.int32, (G, G), 1)).astype(jnp.float32)
    alpha_c = jnp.sum(eye * alpha, axis=1, keepdims=True)
    contrib = lax.dot_general(ohf * e, hp, (((0,), (0,)), ((), ())),
                              precision=lax.Precision.HIGHEST,
                              preferred_element_type=jnp.float32)
    num_acc[...] = num_acc[...] * alpha_c + contrib

    @pl.when(i == nb - 1)
    def _():
        d_c = jnp.sum(eye * d_acc[...], axis=1, keepdims=True)
        out_ref[...] = prev_ref[...] + num_acc[...] / jnp.maximum(d_c, 1e-16)


def _att(h, t, kept, batch, aw, ab, wn, prev, with_xw, bm=1024):
    nb = NP // bm
    out_shape = [jax.ShapeDtypeStruct((G, 128), jnp.float32),
                 jax.ShapeDtypeStruct((NP, 128), jnp.float32)]
    out_specs = [pl.BlockSpec((G, 128), lambda i: (0, 0)),
                 pl.BlockSpec((bm, 128), lambda i: (i, 0))]
    return pl.pallas_call(
        functools.partial(_att_body, nb, with_xw),
        grid=(nb,),
        in_specs=[
            pl.BlockSpec((bm, 128), lambda i: (i, 0)),
            pl.BlockSpec((bm, 1), lambda i: (i, 0)),
            pl.BlockSpec((bm, 1), lambda i: (i, 0)),
            pl.BlockSpec((bm, 1), lambda i: (i, 0)),
            pl.BlockSpec((1, 128), lambda i: (0, 0)),
            pl.BlockSpec((1, 1), lambda i: (0, 0)),
            pl.BlockSpec((128, 128), lambda i: (0, 0)),
            pl.BlockSpec((G, 128), lambda i: (0, 0)),
        ],
        out_specs=out_specs,
        out_shape=out_shape,
        scratch_shapes=[pltpu.VMEM((1, G), jnp.float32),
                        pltpu.VMEM((1, G), jnp.float32),
                        pltpu.VMEM((G, 128), jnp.float32)],
    )(h, t, kept, batch, aw, ab, wn, prev)


def _hist_body(c_hbm, src_hbm, dst_hbm, out_hbm, c_v, src_v, dst_v, acc_v):
    wid = lax.axis_index("s") * NC + lax.axis_index("c")
    pltpu.sync_copy(c_hbm, c_v)
    pltpu.sync_copy(src_hbm.at[wid], src_v)
    pltpu.sync_copy(dst_hbm.at[wid], dst_v)

    def zero(i, _):
        acc_v[pl.ds(i * L, L)] = jnp.zeros((L,), jnp.float32)
        return 0

    lax.fori_loop(0, NP // L, zero, 0)

    def step(i, _):
        s16 = src_v[pl.ds(i * L, L)]
        d16 = dst_v[pl.ds(i * L, L)]
        vals = plsc.load_gather(c_v, [s16])
        plsc.addupdate_scatter(acc_v, [d16], vals)
        return 0

    lax.fori_loop(0, EPT // L, step, 0)
    pltpu.sync_copy(acc_v, out_hbm.at[wid])


@functools.lru_cache(maxsize=None)
def _hist_kernel():
    return pl.kernel(
        _hist_body,
        out_type=jax.ShapeDtypeStruct((NT, NP), jnp.float32),
        mesh=plsc.VectorSubcoreMesh(
            core_axis_name="c", subcore_axis_name="s",
            num_cores=NC, num_subcores=NS),
        compiler_params=pltpu.CompilerParams(needs_layout_passes=False),
        scratch_types=[
            pltpu.VMEM((NP,), jnp.float32),
            pltpu.VMEM((EPT,), jnp.int32),
            pltpu.VMEM((EPT,), jnp.int32),
            pltpu.VMEM((NP,), jnp.float32),
        ],
    )


def _hist_call(c, src2, dst2):
    return _hist_kernel()(c, src2, dst2)


def _edge_body(y_hbm, src_hbm, dst_hbm, out_hbm,
               src_v, dst_v, rows_v, acc_sh, sem):
    cid = lax.axis_index("c")
    sid = lax.axis_index("s")
    tid = cid * NS + sid
    pltpu.sync_copy(src_hbm.at[tid], src_v)
    pltpu.sync_copy(dst_hbm.at[tid], dst_v)

    def zrow(i, _):
        rows_v[i // (128 // L), pl.ds((i % (128 // L)) * L, L)] = (
            jnp.zeros((L,), jnp.float32))
        return 0

    lax.fori_loop(0, CH * (128 // L), zrow, 0)
    for j in range(RPT // CH):
        pltpu.sync_copy(rows_v, acc_sh.at[pl.ds(sid * RPT + j * CH, CH)])
    plsc.subcore_barrier()

    def chunk(j, _):
        pltpu.async_copy(y_hbm.at[src_v.at[j]], rows_v, sem).wait()
        pltpu.sync_copy(rows_v, acc_sh.at[dst_v.at[j]], add=True)
        return 0

    lax.fori_loop(0, CPT, chunk, 0)
    plsc.subcore_barrier()
    for j in range(RPT // CH):
        r0 = sid * RPT + j * CH
        pltpu.sync_copy(acc_sh.at[pl.ds(r0, CH)], rows_v)
        pltpu.sync_copy(rows_v, out_hbm.at[cid, pl.ds(r0, CH)])


@functools.lru_cache(maxsize=None)
def _edge_kernel():
    return pl.kernel(
        _edge_body,
        out_type=jax.ShapeDtypeStruct((NC, NP, 128), jnp.float32),
        mesh=plsc.VectorSubcoreMesh(
            core_axis_name="c", subcore_axis_name="s",
            num_cores=NC, num_subcores=NS),
        compiler_params=pltpu.CompilerParams(needs_layout_passes=False),
        scratch_types=[
            pltpu.VMEM((CPT, CH), jnp.int32),
            pltpu.VMEM((CPT, CH), jnp.int32),
            pltpu.VMEM((CH, 128), jnp.float32),
            pltpu.VMEM_SHARED((NP, 128), jnp.float32),
            pltpu.SemaphoreType.DMA,
        ],
    )


def _edge_call(y, src3, dst3):
    return _edge_kernel()(y, src3, dst3)


def kernel(x, edge_index, batch, lin_W, lin_b, gcl1_W, gcl1_b, pool1_p,
           hgcl0_W, hgcl0_b, pool2_p, att_W, att_b):
    f32 = jnp.float32
    xp = jnp.zeros((NP, FP), f32).at[:N, :F].set(x)
    w1 = jnp.zeros((FP, 256), f32).at[:F].set(lin_W)
    batch_p = jnp.full((NP,), G - 1, jnp.int32).at[:N].set(batch)
    batch_c = batch_p.reshape(1, NP)
    batch_r = batch_p.reshape(NP, 1)
    valid = (jnp.arange(NP) < N).astype(f32).reshape(NP, 1)
    src = jnp.full((EP,), N, jnp.int32).at[:E].set(edge_index[0])
    dst = jnp.full((EP,), N, jnp.int32).at[:E].set(edge_index[1])
    src2 = src.reshape(NT, EPT)
    dst2 = dst.reshape(NT, EPT)
    src3 = src.reshape(NT, CPT, CH)
    dst3 = dst.reshape(NT, CPT, CH)
    p1 = (pool1_p / jnp.maximum(jnp.linalg.norm(pool1_p), 1e-12)).reshape(1, 128)
    p2 = (pool2_p / jnp.maximum(jnp.linalg.norm(pool2_p), 1e-12)).reshape(1, 128)
    aw = att_W.reshape(1, 128)
    ab = att_b.reshape(1, 1)
    zero_out = jnp.zeros((G, 128), f32)

    xw1 = _mm2(xp, w1, lin_b.reshape(1, 256), gcl1_W)
    hist1 = _hist_call(valid[:, 0], src2, dst2)
    a1_2d, sn1_2d = _scale_a(hist1.reshape(NT, NP // 128, 128),
                             valid.reshape(NP // 128, 128))
    a1 = a1_2d.reshape(NP, 1)
    sn1 = sn1_2d.reshape(NP, 1)
    y1 = _scale_y(a1, xw1)
    ep1 = _edge_call(y1, src3, dst3)
    h1, s1, t1, k1 = _conv(ep1, a1, sn1, xw1, gcl1_b.reshape(1, 128), valid,
                           p1, batch_r)
    kept1 = _rank(s1, batch_r, valid, s1.reshape(1, NP), batch_c,
                  valid.reshape(1, NP), k1)
    out1, xw2 = _att(h1, t1, kept1, batch_r, aw, ab, hgcl0_W, zero_out,
                     with_xw=True)

    hist2 = _hist_call(kept1[:, 0], src2, dst2)
    a2_2d, sn2_2d = _scale_a(hist2.reshape(NT, NP // 128, 128),
                             kept1.reshape(NP // 128, 128))
    a2 = a2_2d.reshape(NP, 1)
    sn2 = sn2_2d.reshape(NP, 1)
    y2 = _scale_y(a2, xw2)
    ep2 = _edge_call(y2, src3, dst3)
    h2, s2, t2, k2 = _conv(ep2, a2, sn2, xw2, hgcl0_b.reshape(1, 128), kept1,
                           p2, batch_r)
    kept2 = _rank(s2, batch_r, kept1, s2.reshape(1, NP), batch_c,
                  kept1.reshape(1, NP), k2)
    out, _ = _att(h2, t2, kept2, batch_r, aw, ab, hgcl0_W, out1, with_xw=False)
    return out


# speedup vs baseline: 14.8974x; 1.1146x over previous
import functools

import jax
import jax.numpy as jnp
from jax import lax
from jax.experimental import pallas as pl
from jax.experimental.pallas import tpu as pltpu
from jax.experimental.pallas import tpu_sc as plsc

N = 10000
NP = 10240
E = 320000
G = 16
RATIO = 0.5
F = 129
FP = 136
NC, NS, L = 2, 16, 16
NT = NC * NS
CH = 64
CPT = 158
EPT = CH * CPT
EP = NT * EPT
RPT = NP // NS


def _mm2_body(x_ref, w1_ref, b1_ref, w2_ref, o_ref):
    h = jnp.maximum(
        jnp.dot(x_ref[...], w1_ref[...], preferred_element_type=jnp.float32)
        + b1_ref[...], 0.0)
    o_ref[...] = jnp.dot(h, w2_ref[...], preferred_element_type=jnp.float32)


def _mm2(x, w1, b1, w2, bm=1024):
    nb = NP // bm
    return pl.pallas_call(
        _mm2_body,
        grid=(nb,),
        in_specs=[
            pl.BlockSpec((bm, FP), lambda i: (i, 0)),
            pl.BlockSpec((FP, 256), lambda i: (0, 0)),
            pl.BlockSpec((1, 256), lambda i: (0, 0)),
            pl.BlockSpec((256, 128), lambda i: (0, 0)),
        ],
        out_specs=pl.BlockSpec((bm, 128), lambda i: (i, 0)),
        out_shape=jax.ShapeDtypeStruct((NP, 128), jnp.float32),
    )(x, w1, b1, w2)


def _scale_a_body(hp_ref, nm_ref, a_ref, sn_ref):
    degraw = jnp.sum(hp_ref[...], axis=0)
    nm = nm_ref[...]
    deg = nm * degraw + nm
    pos = deg > 0
    dis = jnp.where(pos, lax.rsqrt(jnp.maximum(deg, 1e-12)), 0.0)
    a_ref[...] = dis * nm
    sn_ref[...] = jnp.where(pos, 1.0 / jnp.maximum(deg, 1e-12), 0.0) * nm


def _scale_a(histp3, nm2d, bm=16):
    nb = (NP // 128) // bm
    return pl.pallas_call(
        _scale_a_body,
        grid=(nb,),
        in_specs=[
            pl.BlockSpec((NT, bm, 128), lambda i: (0, i, 0)),
            pl.BlockSpec((bm, 128), lambda i: (i, 0)),
        ],
        out_specs=[
            pl.BlockSpec((bm, 128), lambda i: (i, 0)),
            pl.BlockSpec((bm, 128), lambda i: (i, 0)),
        ],
        out_shape=[
            jax.ShapeDtypeStruct((NP // 128, 128), jnp.float32),
            jax.ShapeDtypeStruct((NP // 128, 128), jnp.float32),
        ],
    )(histp3, nm2d)


def _scale_y_body(a_ref, xw_ref, y_ref):
    y_ref[...] = a_ref[...] * xw_ref[...]


def _scale_y(a, xw, bm=1024):
    nb = NP // bm
    return pl.pallas_call(
        _scale_y_body,
        grid=(nb,),
        in_specs=[
            pl.BlockSpec((bm, 1), lambda i: (i, 0)),
            pl.BlockSpec((bm, 128), lambda i: (i, 0)),
        ],
        out_specs=pl.BlockSpec((bm, 128), lambda i: (i, 0)),
        out_shape=jax.ShapeDtypeStruct((NP, 128), jnp.float32),
    )(a, xw)


def _conv_body(nb, ep_ref, a_ref, sn_ref, xw_ref, b_ref, nm_ref, p_ref,
               nrm_ref, batch_ref, h_ref, s_ref, t_ref, k_ref, cnt_acc):
    i = pl.program_id(0)

    @pl.when(i == 0)
    def _():
        cnt_acc[...] = jnp.zeros_like(cnt_acc)

    nm = nm_ref[...]
    agg = (ep_ref[0] + ep_ref[1]) * a_ref[...]
    h = jnp.maximum((agg + sn_ref[...] * xw_ref[...] + b_ref[...]) * nm, 0.0)
    h = h * nm
    h_ref[...] = h
    s = jnp.dot(h, p_ref[...], preferred_element_type=jnp.float32) / nrm_ref[...]
    sm = jnp.where(nm > 0, s, -1e30)
    s_ref[...] = sm
    t_ref[...] = jnp.tanh(sm)
    oh = (batch_ref[...] == lax.broadcasted_iota(jnp.int32, (1, G), 1)
          ).astype(jnp.float32) * nm
    cnt_acc[...] += jnp.sum(oh, axis=0, keepdims=True)

    @pl.when(i == nb - 1)
    def _():
        k_ref[...] = jnp.ceil(RATIO * cnt_acc[...])


def _conv(ep, a, sn, xw, b, nm, p_raw, p_nrm, batch, bm=1024):
    nb = NP // bm
    return pl.pallas_call(
        functools.partial(_conv_body, nb),
        grid=(nb,),
        in_specs=[
            pl.BlockSpec((2, bm, 128), lambda i: (0, i, 0)),
            pl.BlockSpec((bm, 1), lambda i: (i, 0)),
            pl.BlockSpec((bm, 1), lambda i: (i, 0)),
            pl.BlockSpec((bm, 128), lambda i: (i, 0)),
            pl.BlockSpec((1, 128), lambda i: (0, 0)),
            pl.BlockSpec((bm, 1), lambda i: (i, 0)),
            pl.BlockSpec((128, 1), lambda i: (0, 0)),
            pl.BlockSpec((1, 1), lambda i: (0, 0)),
            pl.BlockSpec((bm, 1), lambda i: (i, 0)),
        ],
        out_specs=[
            pl.BlockSpec((bm, 128), lambda i: (i, 0)),
            pl.BlockSpec((bm, 1), lambda i: (i, 0)),
            pl.BlockSpec((bm, 1), lambda i: (i, 0)),
            pl.BlockSpec((1, G), lambda i: (0, 0)),
        ],
        out_shape=[
            jax.ShapeDtypeStruct((NP, 128), jnp.float32),
            jax.ShapeDtypeStruct((NP, 1), jnp.float32),
            jax.ShapeDtypeStruct((NP, 1), jnp.float32),
            jax.ShapeDtypeStruct((1, G), jnp.float32),
        ],
        scratch_shapes=[pltpu.VMEM((1, G), jnp.float32)],
    )(ep, a, sn, xw, b, nm, p_raw, p_nrm, batch)


def _rank_body(bm, bn, s_ref, b_ref, al_ref, sc_ref, bc_ref, alc_ref, k_ref,
               kept_ref):
    i = pl.program_id(0)
    sr = s_ref[...]
    br = b_ref[...]
    ar = al_ref[...]
    irow = i * bm + lax.broadcasted_iota(jnp.int32, (bm, 1), 0)
    cnt = jnp.zeros((bm, 1), jnp.float32)
    for cb in range(NP // bn):
        sc = sc_ref[:, cb * bn:(cb + 1) * bn]
        bc = bc_ref[:, cb * bn:(cb + 1) * bn]
        ac = alc_ref[:, cb * bn:(cb + 1) * bn]
        jcol = cb * bn + lax.broadcasted_iota(jnp.int32, (1, bn), 1)
        better = (sc > sr) | ((sc == sr) & (jcol < irow))
        m = better & (bc == br) & (ac > 0)
        cnt = cnt + jnp.sum(m.astype(jnp.float32), axis=1, keepdims=True)
    oh = (br == lax.broadcasted_iota(jnp.int32, (bm, G), 1)).astype(jnp.float32)
    kr = jnp.sum(oh * k_ref[...], axis=1, keepdims=True)
    kept_ref[...] = jnp.where((ar > 0) & (cnt < kr), 1.0, 0.0)


def _rank(s, batch, alive, s_c, b_c, al_c, k, bm=256, bn=512):
    nb = NP // bm
    return pl.pallas_call(
        functools.partial(_rank_body, bm, bn),
        grid=(nb,),
        in_specs=[
            pl.BlockSpec((bm, 1), lambda i: (i, 0)),
            pl.BlockSpec((bm, 1), lambda i: (i, 0)),
            pl.BlockSpec((bm, 1), lambda i: (i, 0)),
            pl.BlockSpec((1, NP), lambda i: (0, 0)),
            pl.BlockSpec((1, NP), lambda i: (0, 0)),
            pl.BlockSpec((1, NP), lambda i: (0, 0)),
            pl.BlockSpec((1, G), lambda i: (0, 0)),
        ],
        out_specs=pl.BlockSpec((bm, 1), lambda i: (i, 0)),
        out_shape=jax.ShapeDtypeStruct((NP, 1), jnp.float32),
    )(s, batch, alive, s_c, b_c, al_c, k)


def _att_body(nb, with_xw, h_ref, t_ref, kept_ref, batch_ref, aw_ref, ab_ref,
              wn_ref, prev_ref, out_ref, xw_ref, m_acc, d_acc, num_acc):
    i = pl.program_id(0)

    @pl.when(i == 0)
    def _():
        m_acc[...] = jnp.full_like(m_acc, -1e30)
        d_acc[...] = jnp.zeros_like(d_acc)
        num_acc[...] = jnp.zeros_like(num_acc)

    kept = kept_ref[...]
    hp = h_ref[...] * t_ref[...] * kept
    if with_xw:
        xw_ref[...] = jnp.dot(hp, wn_ref[...],
                              preferred_element_type=jnp.float32)
    g = jnp.dot(hp, aw_ref[...], preferred_element_type=jnp.float32) \
        + ab_ref[...]
    gm = jnp.where(kept > 0, g, -1e30)
    oh = (batch_ref[...] == lax.broadcasted_iota(jnp.int32, (1, G), 1))
    ohf = oh.astype(jnp.float32)
    bmax = jnp.max(jnp.where(oh, gm, -1e30), axis=0, keepdims=True)
    m_old = m_acc[...]
    m_new = jnp.maximum(m_old, bmax)
    alpha = jnp.exp(m_old - m_new)
    m_acc[...] = m_new
    mn = jnp.sum(ohf * m_new, axis=1, keepdims=True)
    e = jnp.exp(gm - mn) * kept
    d_acc[...] = d_acc[...] * alpha + jnp.sum(ohf * e, axis=0, keepdims=True)
    eye = (lax.broadcasted_iota(jnp.int32, (G, G), 0)
           == lax.broadcasted_iota(jnp.int32, (G, G), 1)).astype(jnp.float32)
    alpha_c = jnp.sum(eye * alpha, axis=1, keepdims=True)
    contrib = lax.dot_general(ohf * e, hp, (((0,), (0,)), ((), ())),
                              precision=lax.Precision.HIGHEST,
                              preferred_element_type=jnp.float32)
    num_acc[...] = num_acc[...] * alpha_c + contrib

    @pl.when(i == nb - 1)
    def _():
        d_c = jnp.sum(eye * d_acc[...], axis=1, keepdims=True)
        out_ref[...] = prev_ref[...] + num_acc[...] / jnp.maximum(d_c, 1e-16)


def _att(h, t, kept, batch, aw, ab, wn, prev, with_xw, bm=1024):
    nb = NP // bm
    out_shape = [jax.ShapeDtypeStruct((G, 128), jnp.float32),
                 jax.ShapeDtypeStruct((NP, 128), jnp.float32)]
    out_specs = [pl.BlockSpec((G, 128), lambda i: (0, 0)),
                 pl.BlockSpec((bm, 128), lambda i: (i, 0))]
    return pl.pallas_call(
        functools.partial(_att_body, nb, with_xw),
        grid=(nb,),
        in_specs=[
            pl.BlockSpec((bm, 128), lambda i: (i, 0)),
            pl.BlockSpec((bm, 1), lambda i: (i, 0)),
            pl.BlockSpec((bm, 1), lambda i: (i, 0)),
            pl.BlockSpec((bm, 1), lambda i: (i, 0)),
            pl.BlockSpec((128, 1), lambda i: (0, 0)),
            pl.BlockSpec((1, 1), lambda i: (0, 0)),
            pl.BlockSpec((128, 128), lambda i: (0, 0)),
            pl.BlockSpec((G, 128), lambda i: (0, 0)),
        ],
        out_specs=out_specs,
        out_shape=out_shape,
        scratch_shapes=[pltpu.VMEM((1, G), jnp.float32),
                        pltpu.VMEM((1, G), jnp.float32),
                        pltpu.VMEM((G, 128), jnp.float32)],
    )(h, t, kept, batch, aw, ab, wn, prev)


def _hist_body(c_hbm, src_hbm, dst_hbm, out_hbm, c_v, src_v, dst_v, acc_v):
    wid = lax.axis_index("s") * NC + lax.axis_index("c")
    pltpu.sync_copy(c_hbm, c_v)
    pltpu.sync_copy(src_hbm.at[wid], src_v)
    pltpu.sync_copy(dst_hbm.at[wid], dst_v)

    def zero(i, _):
        acc_v[pl.ds(i * L, L)] = jnp.zeros((L,), jnp.float32)
        return 0

    lax.fori_loop(0, NP // L, zero, 0)

    def step(i, _):
        s16 = src_v[pl.ds(i * L, L)]
        d16 = dst_v[pl.ds(i * L, L)]
        vals = plsc.load_gather(c_v, [s16])
        plsc.addupdate_scatter(acc_v, [d16], vals)
        return 0

    lax.fori_loop(0, EPT // L, step, 0)
    pltpu.sync_copy(acc_v, out_hbm.at[wid])


@functools.lru_cache(maxsize=None)
def _hist_kernel():
    return pl.kernel(
        _hist_body,
        out_type=jax.ShapeDtypeStruct((NT, NP), jnp.float32),
        mesh=plsc.VectorSubcoreMesh(
            core_axis_name="c", subcore_axis_name="s",
            num_cores=NC, num_subcores=NS),
        compiler_params=pltpu.CompilerParams(needs_layout_passes=False),
        scratch_types=[
            pltpu.VMEM((NP,), jnp.float32),
            pltpu.VMEM((EPT,), jnp.int32),
            pltpu.VMEM((EPT,), jnp.int32),
            pltpu.VMEM((NP,), jnp.float32),
        ],
    )


def _hist_call(c, src2, dst2):
    return _hist_kernel()(c, src2, dst2)


def _edge_body(y_hbm, src_hbm, dst_hbm, out_hbm,
               src_v, dst_v, rows0, rows1, acc_sh, sem0, sem1):
    cid = lax.axis_index("c")
    sid = lax.axis_index("s")
    tid = cid * NS + sid
    pltpu.sync_copy(src_hbm.at[tid], src_v)
    pltpu.sync_copy(dst_hbm.at[tid], dst_v)

    def sidx(j):
        return src_v.at[pl.ds(j * CH, CH)]

    def zrow(i, _):
        rows0[i // (128 // L), pl.ds((i % (128 // L)) * L, L)] = (
            jnp.zeros((L,), jnp.float32))
        return 0

    lax.fori_loop(0, CH * (128 // L), zrow, 0)
    for j in range(RPT // CH):
        pltpu.sync_copy(rows0, acc_sh.at[pl.ds(sid * RPT + j * CH, CH)])
    plsc.subcore_barrier()

    pltpu.async_copy(y_hbm.at[sidx(0)], rows0, sem0)

    def pair(p, _):
        j = 2 * p
        pltpu.make_async_copy(y_hbm.at[sidx(j)], rows0, sem0).wait()
        pltpu.async_copy(y_hbm.at[sidx(j + 1)], rows1, sem1)
        pltpu.sync_copy(rows0, acc_sh.at[dst_v.at[j]], add=True)
        jn = jnp.minimum(j + 2, CPT - 1)
        pltpu.async_copy(y_hbm.at[sidx(jn)], rows0, sem0)
        pltpu.make_async_copy(y_hbm.at[sidx(j + 1)], rows1, sem1).wait()
        pltpu.sync_copy(rows1, acc_sh.at[dst_v.at[j + 1]], add=True)
        return 0

    lax.fori_loop(0, CPT // 2, pair, 0)
    pltpu.make_async_copy(y_hbm.at[sidx(0)], rows0, sem0).wait()
    plsc.subcore_barrier()
    for j in range(RPT // CH):
        r0 = sid * RPT + j * CH
        pltpu.sync_copy(acc_sh.at[pl.ds(r0, CH)], rows0)
        pltpu.sync_copy(rows0, out_hbm.at[cid, pl.ds(r0, CH)])


@functools.lru_cache(maxsize=None)
def _edge_kernel():
    return pl.kernel(
        _edge_body,
        out_type=jax.ShapeDtypeStruct((NC, NP, 128), jnp.float32),
        mesh=plsc.VectorSubcoreMesh(
            core_axis_name="c", subcore_axis_name="s",
            num_cores=NC, num_subcores=NS),
        compiler_params=pltpu.CompilerParams(needs_layout_passes=False),
        scratch_types=[
            pltpu.VMEM((EPT,), jnp.int32),
            pltpu.VMEM((CPT, CH), jnp.int32),
            pltpu.VMEM((CH, 128), jnp.float32),
            pltpu.VMEM((CH, 128), jnp.float32),
            pltpu.VMEM_SHARED((NP, 128), jnp.float32),
            pltpu.SemaphoreType.DMA,
            pltpu.SemaphoreType.DMA,
        ],
    )


def _edge_call(y, src2, dst3):
    return _edge_kernel()(y, src2, dst3)


def kernel(x, edge_index, batch, lin_W, lin_b, gcl1_W, gcl1_b, pool1_p,
           hgcl0_W, hgcl0_b, pool2_p, att_W, att_b):
    f32 = jnp.float32
    xp = jnp.zeros((NP, FP), f32).at[:N, :F].set(x)
    w1 = jnp.zeros((FP, 256), f32).at[:F].set(lin_W)
    batch_p = jnp.full((NP,), G - 1, jnp.int32).at[:N].set(batch)
    batch_c = batch_p.reshape(1, NP)
    batch_r = batch_p.reshape(NP, 1)
    valid = (jnp.arange(NP) < N).astype(f32).reshape(NP, 1)
    src = jnp.full((EP,), N, jnp.int32).at[:E].set(edge_index[0])
    dst = jnp.full((EP,), N, jnp.int32).at[:E].set(edge_index[1])
    src2 = src.reshape(NT, EPT)
    dst2 = dst.reshape(NT, EPT)
    dst3 = dst.reshape(NT, CPT, CH)
    p1 = pool1_p.reshape(128, 1)
    n1 = jnp.maximum(jnp.linalg.norm(pool1_p), 1e-12).reshape(1, 1)
    p2 = pool2_p.reshape(128, 1)
    n2 = jnp.maximum(jnp.linalg.norm(pool2_p), 1e-12).reshape(1, 1)
    aw = att_W.reshape(128, 1)
    ab = att_b.reshape(1, 1)
    zero_out = jnp.zeros((G, 128), f32)

    xw1 = _mm2(xp, w1, lin_b.reshape(1, 256), gcl1_W)
    hist1 = _hist_call(valid[:, 0], src2, dst2)
    a1_2d, sn1_2d = _scale_a(hist1.reshape(NT, NP // 128, 128),
                             valid.reshape(NP // 128, 128))
    a1 = a1_2d.reshape(NP, 1)
    sn1 = sn1_2d.reshape(NP, 1)
    y1 = _scale_y(a1, xw1)
    ep1 = _edge_call(y1, src2, dst3)
    h1, s1, t1, k1 = _conv(ep1, a1, sn1, xw1, gcl1_b.reshape(1, 128), valid,
                           p1, n1, batch_r)
    kept1 = _rank(s1, batch_r, valid, s1.reshape(1, NP), batch_c,
                  valid.reshape(1, NP), k1)
    out1, xw2 = _att(h1, t1, kept1, batch_r, aw, ab, hgcl0_W, zero_out,
                     with_xw=True)

    hist2 = _hist_call(kept1[:, 0], src2, dst2)
    a2_2d, sn2_2d = _scale_a(hist2.reshape(NT, NP // 128, 128),
                             kept1.reshape(NP // 128, 128))
    a2 = a2_2d.reshape(NP, 1)
    sn2 = sn2_2d.reshape(NP, 1)
    y2 = _scale_y(a2, xw2)
    ep2 = _edge_call(y2, src2, dst3)
    h2, s2, t2, k2 = _conv(ep2, a2, sn2, xw2, hgcl0_b.reshape(1, 128), kept1,
                           p2, n2, batch_r)
    kept2 = _rank(s2, batch_r, kept1, s2.reshape(1, NP), batch_c,
                  kept1.reshape(1, NP), k2)
    out, _ = _att(h2, t2, kept2, batch_r, aw, ab, hgcl0_W, out1, with_xw=False)
    return out


# speedup vs baseline: 20.7277x; 1.3914x over previous
import functools

import jax
import jax.numpy as jnp
from jax import lax
from jax.experimental import pallas as pl
from jax.experimental.pallas import tpu as pltpu
from jax.experimental.pallas import tpu_sc as plsc

N = 10000
NP = 10240
E = 320000
G = 16
RATIO = 0.5
F = 129
FP = 136
NC, NS, L = 2, 16, 16
NT = NC * NS
CH = 64
CPT = 158
EPT = CH * CPT
EP = NT * EPT
RPT = NP // NS


def _mm2_body(x_ref, w1_ref, b1_ref, w2_ref, o_ref):
    h = jnp.maximum(
        jnp.dot(x_ref[...], w1_ref[...], preferred_element_type=jnp.float32)
        + b1_ref[...], 0.0)
    o_ref[...] = jnp.dot(h, w2_ref[...], preferred_element_type=jnp.float32)


def _mm2(x, w1, b1, w2, bm=1024):
    nb = NP // bm
    return pl.pallas_call(
        _mm2_body,
        grid=(nb,),
        in_specs=[
            pl.BlockSpec((bm, FP), lambda i: (i, 0)),
            pl.BlockSpec((FP, 256), lambda i: (0, 0)),
            pl.BlockSpec((1, 256), lambda i: (0, 0)),
            pl.BlockSpec((256, 128), lambda i: (0, 0)),
        ],
        out_specs=pl.BlockSpec((bm, 128), lambda i: (i, 0)),
        out_shape=jax.ShapeDtypeStruct((NP, 128), jnp.float32),
    )(x, w1, b1, w2)


def _scale_a_body(hp_ref, nm_ref, a_ref, sn_ref):
    degraw = jnp.sum(hp_ref[...], axis=0)
    nm = nm_ref[...]
    deg = nm * degraw + nm
    pos = deg > 0
    dis = jnp.where(pos, lax.rsqrt(jnp.maximum(deg, 1e-12)), 0.0)
    a_ref[...] = dis * nm
    sn_ref[...] = jnp.where(pos, 1.0 / jnp.maximum(deg, 1e-12), 0.0) * nm


def _scale_a(histp3, nm2d, bm=16):
    nb = (NP // 128) // bm
    return pl.pallas_call(
        _scale_a_body,
        grid=(nb,),
        in_specs=[
            pl.BlockSpec((NT, bm, 128), lambda i: (0, i, 0)),
            pl.BlockSpec((bm, 128), lambda i: (i, 0)),
        ],
        out_specs=[
            pl.BlockSpec((bm, 128), lambda i: (i, 0)),
            pl.BlockSpec((bm, 128), lambda i: (i, 0)),
        ],
        out_shape=[
            jax.ShapeDtypeStruct((NP // 128, 128), jnp.float32),
            jax.ShapeDtypeStruct((NP // 128, 128), jnp.float32),
        ],
    )(histp3, nm2d)


def _scale_y_body(a_ref, xw_ref, y_ref):
    y_ref[...] = a_ref[...] * xw_ref[...]


def _scale_y(a, xw, bm=1024):
    nb = NP // bm
    return pl.pallas_call(
        _scale_y_body,
        grid=(nb,),
        in_specs=[
            pl.BlockSpec((bm, 1), lambda i: (i, 0)),
            pl.BlockSpec((bm, 128), lambda i: (i, 0)),
        ],
        out_specs=pl.BlockSpec((bm, 128), lambda i: (i, 0)),
        out_shape=jax.ShapeDtypeStruct((NP, 128), jnp.float32),
    )(a, xw)


def _conv_body(nb, ep_ref, a_ref, sn_ref, xw_ref, b_ref, nm_ref, p_ref,
               nrm_ref, batch_ref, h_ref, s_ref, t_ref, k_ref, cnt_acc):
    i = pl.program_id(0)

    @pl.when(i == 0)
    def _():
        cnt_acc[...] = jnp.zeros_like(cnt_acc)

    nm = nm_ref[...]
    agg = (ep_ref[0] + ep_ref[1]) * a_ref[...]
    h = jnp.maximum((agg + sn_ref[...] * xw_ref[...] + b_ref[...]) * nm, 0.0)
    h = h * nm
    h_ref[...] = h
    s = jnp.dot(h, p_ref[...], preferred_element_type=jnp.float32) / nrm_ref[...]
    sm = jnp.where(nm > 0, s, -1e30)
    s_ref[...] = sm
    t_ref[...] = jnp.tanh(sm)
    oh = (batch_ref[...] == lax.broadcasted_iota(jnp.int32, (1, G), 1)
          ).astype(jnp.float32) * nm
    cnt_acc[...] += jnp.sum(oh, axis=0, keepdims=True)

    @pl.when(i == nb - 1)
    def _():
        k_ref[...] = jnp.ceil(RATIO * cnt_acc[...])


def _conv(ep, a, sn, xw, b, nm, p_raw, p_nrm, batch, bm=1024):
    nb = NP // bm
    return pl.pallas_call(
        functools.partial(_conv_body, nb),
        grid=(nb,),
        in_specs=[
            pl.BlockSpec((2, bm, 128), lambda i: (0, i, 0)),
            pl.BlockSpec((bm, 1), lambda i: (i, 0)),
            pl.BlockSpec((bm, 1), lambda i: (i, 0)),
            pl.BlockSpec((bm, 128), lambda i: (i, 0)),
            pl.BlockSpec((1, 128), lambda i: (0, 0)),
            pl.BlockSpec((bm, 1), lambda i: (i, 0)),
            pl.BlockSpec((128, 1), lambda i: (0, 0)),
            pl.BlockSpec((1, 1), lambda i: (0, 0)),
            pl.BlockSpec((bm, 1), lambda i: (i, 0)),
        ],
        out_specs=[
            pl.BlockSpec((bm, 128), lambda i: (i, 0)),
            pl.BlockSpec((bm, 1), lambda i: (i, 0)),
            pl.BlockSpec((bm, 1), lambda i: (i, 0)),
            pl.BlockSpec((1, G), lambda i: (0, 0)),
        ],
        out_shape=[
            jax.ShapeDtypeStruct((NP, 128), jnp.float32),
            jax.ShapeDtypeStruct((NP, 1), jnp.float32),
            jax.ShapeDtypeStruct((NP, 1), jnp.float32),
            jax.ShapeDtypeStruct((1, G), jnp.float32),
        ],
        scratch_shapes=[pltpu.VMEM((1, G), jnp.float32)],
    )(ep, a, sn, xw, b, nm, p_raw, p_nrm, batch)


def _rank_body(bm, bn, s_ref, b_ref, al_ref, sc_ref, bc_ref, alc_ref, k_ref,
               lo_ref, hi_ref, kept_ref, cnt_ref):
    i = pl.program_id(0)
    sr = s_ref[...]
    br = b_ref[...]
    ar = al_ref[...]
    irow = i * bm + lax.broadcasted_iota(jnp.int32, (bm, 1), 0)
    cnt_ref[...] = jnp.zeros((bm, 1), jnp.float32)
    lo = lo_ref[i]
    hi = hi_ref[i]
    for cb in range(NP // bn):
        @pl.when((cb >= lo) & (cb <= hi))
        def _():
            sc = sc_ref[:, cb * bn:(cb + 1) * bn]
            bc = bc_ref[:, cb * bn:(cb + 1) * bn]
            ac = alc_ref[:, cb * bn:(cb + 1) * bn]
            jcol = cb * bn + lax.broadcasted_iota(jnp.int32, (1, bn), 1)
            better = (sc > sr) | ((sc == sr) & (jcol < irow))
            m = better & (bc == br) & (ac > 0)
            cnt_ref[...] += jnp.sum(m.astype(jnp.float32), axis=1,
                                    keepdims=True)
    oh = (br == lax.broadcasted_iota(jnp.int32, (bm, G), 1)).astype(jnp.float32)
    kr = jnp.sum(oh * k_ref[...], axis=1, keepdims=True)
    kept_ref[...] = jnp.where((ar > 0) & (cnt_ref[...] < kr), 1.0, 0.0)


def _rank(s, batch, alive, s_c, b_c, al_c, k, cb_lo, cb_hi, bm=256, bn=512):
    nb = NP // bm
    return pl.pallas_call(
        functools.partial(_rank_body, bm, bn),
        grid=(nb,),
        in_specs=[
            pl.BlockSpec((bm, 1), lambda i: (i, 0)),
            pl.BlockSpec((bm, 1), lambda i: (i, 0)),
            pl.BlockSpec((bm, 1), lambda i: (i, 0)),
            pl.BlockSpec((1, NP), lambda i: (0, 0)),
            pl.BlockSpec((1, NP), lambda i: (0, 0)),
            pl.BlockSpec((1, NP), lambda i: (0, 0)),
            pl.BlockSpec((1, G), lambda i: (0, 0)),
            pl.BlockSpec((NP // bm,), lambda i: (0,), memory_space=pltpu.SMEM),
            pl.BlockSpec((NP // bm,), lambda i: (0,), memory_space=pltpu.SMEM),
        ],
        out_specs=pl.BlockSpec((bm, 1), lambda i: (i, 0)),
        out_shape=jax.ShapeDtypeStruct((NP, 1), jnp.float32),
        scratch_shapes=[pltpu.VMEM((bm, 1), jnp.float32)],
    )(s, batch, alive, s_c, b_c, al_c, k, cb_lo, cb_hi)


def _att_body(nb, with_xw, h_ref, t_ref, kept_ref, batch_ref, aw_ref, ab_ref,
              wn_ref, prev_ref, out_ref, xw_ref, m_acc, d_acc, num_acc):
    i = pl.program_id(0)

    @pl.when(i == 0)
    def _():
        m_acc[...] = jnp.full_like(m_acc, -1e30)
        d_acc[...] = jnp.zeros_like(d_acc)
        num_acc[...] = jnp.zeros_like(num_acc)

    kept = kept_ref[...]
    hp = h_ref[...] * t_ref[...] * kept
    if with_xw:
        xw_ref[...] = jnp.dot(hp, wn_ref[...],
                              preferred_element_type=jnp.float32)
    g = jnp.dot(hp, aw_ref[...], preferred_element_type=jnp.float32) \
        + ab_ref[...]
    gm = jnp.where(kept > 0, g, -1e30)
    oh = (batch_ref[...] == lax.broadcasted_iota(jnp.int32, (1, G), 1))
    ohf = oh.astype(jnp.float32)
    bmax = jnp.max(jnp.where(oh, gm, -1e30), axis=0, keepdims=True)
    m_old = m_acc[...]
    m_new = jnp.maximum(m_old, bmax)
    alpha = jnp.exp(m_old - m_new)
    m_acc[...] = m_new
    mn = jnp.sum(ohf * m_new, axis=1, keepdims=True)
    e = jnp.exp(gm - mn) * kept
    d_acc[...] = d_acc[...] * alpha + jnp.sum(ohf * e, axis=0, keepdims=True)
    eye = (lax.broadcasted_iota(jnp.int32, (G, G), 0)
           == lax.broadcasted_iota(jnp.int32, (G, G), 1)).astype(jnp.float32)
    alpha_c = jnp.sum(eye * alpha, axis=1, keepdims=True)
    contrib = lax.dot_general(ohf * e, hp, (((0,), (0,)), ((), ())),
                              precision=lax.Precision.HIGHEST,
                              preferred_element_type=jnp.float32)
    num_acc[...] = num_acc[...] * alpha_c + contrib

    @pl.when(i == nb - 1)
    def _():
        d_c = jnp.sum(eye * d_acc[...], axis=1, keepdims=True)
        out_ref[...] = prev_ref[...] + num_acc[...] / jnp.maximum(d_c, 1e-16)


def _att(h, t, kept, batch, aw, ab, wn, prev, with_xw, bm=1024):
    nb = NP // bm
    out_shape = [jax.ShapeDtypeStruct((G, 128), jnp.float32),
                 jax.ShapeDtypeStruct((NP, 128), jnp.float32)]
    out_specs = [pl.BlockSpec((G, 128), lambda i: (0, 0)),
                 pl.BlockSpec((bm, 128), lambda i: (i, 0))]
    return pl.pallas_call(
        functools.partial(_att_body, nb, with_xw),
        grid=(nb,),
        in_specs=[
            pl.BlockSpec((bm, 128), lambda i: (i, 0)),
            pl.BlockSpec((bm, 1), lambda i: (i, 0)),
            pl.BlockSpec((bm, 1), lambda i: (i, 0)),
            pl.BlockSpec((bm, 1), lambda i: (i, 0)),
            pl.BlockSpec((128, 1), lambda i: (0, 0)),
            pl.BlockSpec((1, 1), lambda i: (0, 0)),
            pl.BlockSpec((128, 128), lambda i: (0, 0)),
            pl.BlockSpec((G, 128), lambda i: (0, 0)),
        ],
        out_specs=out_specs,
        out_shape=out_shape,
        scratch_shapes=[pltpu.VMEM((1, G), jnp.float32),
                        pltpu.VMEM((1, G), jnp.float32),
                        pltpu.VMEM((G, 128), jnp.float32)],
    )(h, t, kept, batch, aw, ab, wn, prev)


def _hist_body(c_hbm, src_hbm, dst_hbm, out_hbm, c_v, src_v, dst_v, acc_v):
    wid = lax.axis_index("s") * NC + lax.axis_index("c")
    pltpu.sync_copy(c_hbm, c_v)
    pltpu.sync_copy(src_hbm.at[wid], src_v)
    pltpu.sync_copy(dst_hbm.at[wid], dst_v)

    def zero(i, _):
        acc_v[pl.ds(i * L, L)] = jnp.zeros((L,), jnp.float32)
        return 0

    lax.fori_loop(0, NP // L, zero, 0)

    def step(i, _):
        s16 = src_v[pl.ds(i * L, L)]
        d16 = dst_v[pl.ds(i * L, L)]
        vals = plsc.load_gather(c_v, [s16])
        plsc.addupdate_scatter(acc_v, [d16], vals)
        return 0

    lax.fori_loop(0, EPT // L, step, 0)
    pltpu.sync_copy(acc_v, out_hbm.at[wid])


@functools.lru_cache(maxsize=None)
def _hist_kernel():
    return pl.kernel(
        _hist_body,
        out_type=jax.ShapeDtypeStruct((NT, NP), jnp.float32),
        mesh=plsc.VectorSubcoreMesh(
            core_axis_name="c", subcore_axis_name="s",
            num_cores=NC, num_subcores=NS),
        compiler_params=pltpu.CompilerParams(needs_layout_passes=False),
        scratch_types=[
            pltpu.VMEM((NP,), jnp.float32),
            pltpu.VMEM((EPT,), jnp.int32),
            pltpu.VMEM((EPT,), jnp.int32),
            pltpu.VMEM((NP,), jnp.float32),
        ],
    )


def _hist_call(c, src2, dst2):
    return _hist_kernel()(c, src2, dst2)


def _edge_body(y_hbm, src_hbm, dst_hbm, out_hbm,
               src_v, dst_v, rows0, rows1, acc_sh, sem0, sem1):
    cid = lax.axis_index("c")
    sid = lax.axis_index("s")
    tid = cid * NS + sid
    pltpu.sync_copy(src_hbm.at[tid], src_v)
    pltpu.sync_copy(dst_hbm.at[tid], dst_v)

    def sidx(j):
        return src_v.at[pl.ds(j * CH, CH)]

    def zrow(i, _):
        rows0[i // (128 // L), pl.ds((i % (128 // L)) * L, L)] = (
            jnp.zeros((L,), jnp.float32))
        return 0

    lax.fori_loop(0, CH * (128 // L), zrow, 0)
    for j in range(RPT // CH):
        pltpu.sync_copy(rows0, acc_sh.at[pl.ds(sid * RPT + j * CH, CH)])
    plsc.subcore_barrier()

    pltpu.async_copy(y_hbm.at[sidx(0)], rows0, sem0)

    def pair(p, _):
        j = 2 * p
        pltpu.make_async_copy(y_hbm.at[sidx(j)], rows0, sem0).wait()
        pltpu.async_copy(y_hbm.at[sidx(j + 1)], rows1, sem1)
        pltpu.sync_copy(rows0, acc_sh.at[dst_v.at[j]], add=True)
        jn = jnp.minimum(j + 2, CPT - 1)
        pltpu.async_copy(y_hbm.at[sidx(jn)], rows0, sem0)
        pltpu.make_async_copy(y_hbm.at[sidx(j + 1)], rows1, sem1).wait()
        pltpu.sync_copy(rows1, acc_sh.at[dst_v.at[j + 1]], add=True)
        return 0

    lax.fori_loop(0, CPT // 2, pair, 0)
    pltpu.make_async_copy(y_hbm.at[sidx(0)], rows0, sem0).wait()
    plsc.subcore_barrier()
    for j in range(RPT // CH):
        r0 = sid * RPT + j * CH
        pltpu.sync_copy(acc_sh.at[pl.ds(r0, CH)], rows0)
        pltpu.sync_copy(rows0, out_hbm.at[cid, pl.ds(r0, CH)])


@functools.lru_cache(maxsize=None)
def _edge_kernel():
    return pl.kernel(
        _edge_body,
        out_type=jax.ShapeDtypeStruct((NC, NP, 128), jnp.float32),
        mesh=plsc.VectorSubcoreMesh(
            core_axis_name="c", subcore_axis_name="s",
            num_cores=NC, num_subcores=NS),
        compiler_params=pltpu.CompilerParams(needs_layout_passes=False),
        scratch_types=[
            pltpu.VMEM((EPT,), jnp.int32),
            pltpu.VMEM((CPT, CH), jnp.int32),
            pltpu.VMEM((CH, 128), jnp.float32),
            pltpu.VMEM((CH, 128), jnp.float32),
            pltpu.VMEM_SHARED((NP, 128), jnp.float32),
            pltpu.SemaphoreType.DMA,
            pltpu.SemaphoreType.DMA,
        ],
    )


def _edge_call(y, src2, dst3):
    return _edge_kernel()(y, src2, dst3)


def kernel(x, edge_index, batch, lin_W, lin_b, gcl1_W, gcl1_b, pool1_p,
           hgcl0_W, hgcl0_b, pool2_p, att_W, att_b):
    f32 = jnp.float32
    xp = jnp.zeros((NP, FP), f32).at[:N, :F].set(x)
    w1 = jnp.zeros((FP, 256), f32).at[:F].set(lin_W)
    batch_p = jnp.full((NP,), G - 1, jnp.int32).at[:N].set(batch)
    batch_c = batch_p.reshape(1, NP)
    batch_r = batch_p.reshape(NP, 1)
    valid = (jnp.arange(NP) < N).astype(f32).reshape(NP, 1)
    src = jnp.full((EP,), N, jnp.int32).at[:E].set(edge_index[0])
    dst = jnp.full((EP,), N, jnp.int32).at[:E].set(edge_index[1])
    src2 = src.reshape(NT, EPT)
    dst2 = dst.reshape(NT, EPT)
    dst3 = dst.reshape(NT, CPT, CH)
    p1 = pool1_p.reshape(128, 1)
    n1 = jnp.maximum(jnp.linalg.norm(pool1_p), 1e-12).reshape(1, 1)
    p2 = pool2_p.reshape(128, 1)
    n2 = jnp.maximum(jnp.linalg.norm(pool2_p), 1e-12).reshape(1, 1)
    aw = att_W.reshape(128, 1)
    ab = att_b.reshape(1, 1)
    zero_out = jnp.zeros((G, 128), f32)
    rb_bm, rk_bn = 256, 512
    gids = jnp.arange(G, dtype=batch_p.dtype)
    starts_g = jnp.searchsorted(batch_p, gids, side="left").astype(jnp.int32)
    ends_g = jnp.searchsorted(batch_p, gids, side="right").astype(jnp.int32)
    rb_idx = jnp.arange(NP // rb_bm) * rb_bm
    cb_lo = (starts_g[batch_p[rb_idx]] // rk_bn).astype(jnp.int32)
    cb_hi = jnp.maximum(
        ((ends_g[batch_p[rb_idx + rb_bm - 1]] - 1) // rk_bn).astype(jnp.int32),
        cb_lo)

    xw1 = _mm2(xp, w1, lin_b.reshape(1, 256), gcl1_W)
    hist1 = _hist_call(valid[:, 0], src2, dst2)
    a1_2d, sn1_2d = _scale_a(hist1.reshape(NT, NP // 128, 128),
                             valid.reshape(NP // 128, 128))
    a1 = a1_2d.reshape(NP, 1)
    sn1 = sn1_2d.reshape(NP, 1)
    y1 = _scale_y(a1, xw1)
    ep1 = _edge_call(y1, src2, dst3)
    h1, s1, t1, k1 = _conv(ep1, a1, sn1, xw1, gcl1_b.reshape(1, 128), valid,
                           p1, n1, batch_r)
    kept1 = _rank(s1, batch_r, valid, s1.reshape(1, NP), batch_c,
                  valid.reshape(1, NP), k1, cb_lo, cb_hi)
    out1, xw2 = _att(h1, t1, kept1, batch_r, aw, ab, hgcl0_W, zero_out,
                     with_xw=True)

    hist2 = _hist_call(kept1[:, 0], src2, dst2)
    a2_2d, sn2_2d = _scale_a(hist2.reshape(NT, NP // 128, 128),
                             kept1.reshape(NP // 128, 128))
    a2 = a2_2d.reshape(NP, 1)
    sn2 = sn2_2d.reshape(NP, 1)
    y2 = _scale_y(a2, xw2)
    ep2 = _edge_call(y2, src2, dst3)
    h2, s2, t2, k2 = _conv(ep2, a2, sn2, xw2, hgcl0_b.reshape(1, 128), kept1,
                           p2, n2, batch_r)
    kept2 = _rank(s2, batch_r, kept1, s2.reshape(1, NP), batch_c,
                  kept1.reshape(1, NP), k2, cb_lo, cb_hi)
    out, _ = _att(h2, t2, kept2, batch_r, aw, ab, hgcl0_W, out1, with_xw=False)
    return out
